# baseline (speedup 1.0000x reference)
import functools

import jax
import jax.numpy as jnp
from jax import lax
from jax.experimental import pallas as pl
from jax.experimental.pallas import tpu as pltpu
from jax.experimental.pallas import tpu_sc as plsc

B = 16384
D = 32

_NC, _NS = 2, 16
_NW = _NC * _NS
_BPW = B // _NW
_G = 16


def _sc_gather(user_idx, item_idx, eugT, eumT, eigT, eimT):
  mesh = plsc.VectorSubcoreMesh(core_axis_name="c", subcore_axis_name="s")

  col_t = jax.ShapeDtypeStruct((D, B), jnp.float32)

  @functools.partial(
      pl.kernel,
      mesh=mesh,
      out_type=[col_t, col_t, col_t, col_t],
      compiler_params=pltpu.CompilerParams(use_tc_tiling_on_sc=False, needs_layout_passes=False),
      scratch_types=[
          pltpu.VMEM((_BPW,), jnp.int32),
          pltpu.VMEM((_BPW,), jnp.int32),
          pltpu.VMEM((D, 8 * _G), jnp.float32),
          pltpu.VMEM((D, 8 * _G), jnp.float32),
          pltpu.VMEM((D, 8 * _G), jnp.float32),
          pltpu.VMEM((D, 8 * _G), jnp.float32),
          pltpu.VMEM((D, _BPW), jnp.float32),
          pltpu.VMEM((D, _BPW), jnp.float32),
          pltpu.VMEM((D, _BPW), jnp.float32),
          pltpu.VMEM((D, _BPW), jnp.float32),
          pltpu.SemaphoreType.DMA,
      ],
  )
  def k(uidx_hbm, iidx_hbm, eug_hbm, eum_hbm, eig_hbm, eim_hbm,
        ug_o, um_o, ig_o, im_o,
        uidx_v, iidx_v, ug_st, um_st, ig_st, im_st,
        ug_v, um_v, ig_v, im_v, sem):
    wid = lax.axis_index("s") * _NC + lax.axis_index("c")
    base = wid * _BPW
    pltpu.sync_copy(uidx_hbm.at[pl.ds(base, _BPW)], uidx_v)
    pltpu.sync_copy(iidx_hbm.at[pl.ds(base, _BPW)], iidx_v)

    iota16 = lax.broadcasted_iota(jnp.int32, (16,), 0)

    def body(g):
      uvec = uidx_v[pl.ds(g * _G, _G)]
      ivec = iidx_v[pl.ds(g * _G, _G)]
      copies = []
      for l in range(_G):
        v8 = (uvec[l] // 8) * 8
        w8 = (ivec[l] // 8) * 8
        dst = pl.ds(l * 8, 8)
        copies.append(
            pltpu.async_copy(eug_hbm.at[:, pl.ds(v8, 8)], ug_st.at[:, dst],
                             sem))
        copies.append(
            pltpu.async_copy(eum_hbm.at[:, pl.ds(v8, 8)], um_st.at[:, dst],
                             sem))
        copies.append(
            pltpu.async_copy(eig_hbm.at[:, pl.ds(w8, 8)], ig_st.at[:, dst],
                             sem))
        copies.append(
            pltpu.async_copy(eim_hbm.at[:, pl.ds(w8, 8)], im_st.at[:, dst],
                             sem))
      for cp in copies:
        cp.wait()
      ucol = iota16 * 8 + (uvec & 7)
      icol = iota16 * 8 + (ivec & 7)
      osl = pl.ds(g * _G, _G)
      for f in range(D):
        frow = jnp.full((16,), f, jnp.int32)
        ug_v[f, osl] = plsc.load_gather(ug_st, [frow, ucol])
        um_v[f, osl] = plsc.load_gather(um_st, [frow, ucol])
        ig_v[f, osl] = plsc.load_gather(ig_st, [frow, icol])
        im_v[f, osl] = plsc.load_gather(im_st, [frow, icol])

    pl.loop(0, _BPW // _G)(body)

    out_sl = pl.ds(base, _BPW)
    pltpu.sync_copy(ug_v, ug_o.at[:, out_sl])
    pltpu.sync_copy(um_v, um_o.at[:, out_sl])
    pltpu.sync_copy(ig_v, ig_o.at[:, out_sl])
    pltpu.sync_copy(im_v, im_o.at[:, out_sl])

  return k(user_idx, item_idx, eugT, eumT, eigT, eimT)


def _tc_dense_body(ug_r, ig_r, um_r, im_r, gw_r, gb_r, w1aT_r, w1bT_r, b1_r,
                   w2T_r, b2_r, w3T_r, b3_r, w4_r, b4_r, out_r):
  gmf_logit = jnp.sum(ug_r[...] * ig_r[...] * gw_r[...], axis=0,
                      keepdims=True) + gb_r[0, 0]
  h = jnp.maximum(
      jnp.dot(w1aT_r[...], um_r[...], preferred_element_type=jnp.float32)
      + jnp.dot(w1bT_r[...], im_r[...], preferred_element_type=jnp.float32)
      + b1_r[...], 0.0)
  h = jnp.maximum(
      jnp.dot(w2T_r[...], h, preferred_element_type=jnp.float32) + b2_r[...],
      0.0)
  h = jnp.maximum(
      jnp.dot(w3T_r[...], h, preferred_element_type=jnp.float32) + b3_r[...],
      0.0)
  mlp_logit = jnp.sum(h * w4_r[...], axis=0, keepdims=True) + b4_r[0, 0]
  out_r[...] = 0.5 * (jax.nn.sigmoid(gmf_logit) + jax.nn.sigmoid(mlp_logit))


def kernel(user_indices, item_indices, emb_user_gmf, emb_user_mlp,
           emb_item_gmf, emb_item_mlp, gmf_w, gmf_b, w1, b1, w2, b2, w3, b3,
           w4, b4):
  uidx = jnp.asarray(user_indices, jnp.int32)
  iidx = jnp.asarray(item_indices, jnp.int32)

  ug, um, ig, im = _sc_gather(uidx, iidx, emb_user_gmf.T, emb_user_mlp.T,
                              emb_item_gmf.T, emb_item_mlp.T)

  gw = gmf_w.reshape(D, 1)
  gb = gmf_b.reshape(1, 1)
  w1aT = w1[:D].T
  w1bT = w1[D:].T
  b1c = b1.reshape(-1, 1)
  w2T = w2.T
  b2c = b2.reshape(-1, 1)
  w3T = w3.T
  b3c = b3.reshape(-1, 1)
  w4c = w4.reshape(-1, 1)
  b4c = b4.reshape(1, 1)

  blk = 4096
  grid = B // blk

  def col_spec():
    return pl.BlockSpec((D, blk), lambda i: (0, i))

  def full_spec(shape):
    return pl.BlockSpec(shape, lambda i: tuple(0 for _ in shape))

  out = pl.pallas_call(
      _tc_dense_body,
      grid=(grid,),
      in_specs=[
          col_spec(), col_spec(), col_spec(), col_spec(),
          full_spec(gw.shape), full_spec(gb.shape),
          full_spec(w1aT.shape), full_spec(w1bT.shape), full_spec(b1c.shape),
          full_spec(w2T.shape), full_spec(b2c.shape),
          full_spec(w3T.shape), full_spec(b3c.shape),
          full_spec(w4c.shape), full_spec(b4c.shape),
      ],
      out_specs=pl.BlockSpec((1, blk), lambda i: (0, i)),
      out_shape=jax.ShapeDtypeStruct((1, B), jnp.float32),
  )(ug, ig, um, im, gw, gb, w1aT, w1bT, b1c, w2T, b2c, w3T, b3c, w4c, b4c)
  return out.reshape(B, 1)


# speedup vs baseline: 4.5502x; 4.5502x over previous
import functools

import jax
import jax.numpy as jnp
from jax import lax
from jax.experimental import pallas as pl
from jax.experimental.pallas import tpu as pltpu
from jax.experimental.pallas import tpu_sc as plsc

B = 16384
D = 32

_NC, _NS = 2, 16
_NW = _NC * _NS
_BPW = B // _NW
_CHUNK = 128
_NCH = _BPW // _CHUNK


def _sc_gather(user_idx2d, item_idx2d, eug, eum, eig, eim):
  mesh = plsc.VectorSubcoreMesh(core_axis_name="c", subcore_axis_name="s")

  row_t = jax.ShapeDtypeStruct((B, D), jnp.bfloat16)

  @functools.partial(
      pl.kernel,
      mesh=mesh,
      out_type=[row_t, row_t, row_t, row_t],
      compiler_params=pltpu.CompilerParams(use_tc_tiling_on_sc=False),
      scratch_types=[
          pltpu.VMEM((_NCH, _CHUNK), jnp.int32),
          pltpu.VMEM((_NCH, _CHUNK), jnp.int32),
          pltpu.VMEM((_BPW, D), jnp.bfloat16),
          pltpu.VMEM((_BPW, D), jnp.bfloat16),
          pltpu.VMEM((_BPW, D), jnp.bfloat16),
          pltpu.VMEM((_BPW, D), jnp.bfloat16),
          pltpu.SemaphoreType.DMA,
      ],
  )
  def k(uidx_hbm, iidx_hbm, eug_hbm, eum_hbm, eig_hbm, eim_hbm,
        ug_o, um_o, ig_o, im_o,
        uidx_v, iidx_v, ug_v, um_v, ig_v, im_v, sem):
    wid = lax.axis_index("s") * _NC + lax.axis_index("c")
    base = wid * _BPW
    crow = wid * _NCH
    pltpu.sync_copy(uidx_hbm.at[pl.ds(crow, _NCH)], uidx_v)
    pltpu.sync_copy(iidx_hbm.at[pl.ds(crow, _NCH)], iidx_v)
    copies = []
    for c in range(_NCH):
      sl = pl.ds(c * _CHUNK, _CHUNK)
      copies.append(pltpu.async_copy(eug_hbm.at[uidx_v.at[c]], ug_v.at[sl], sem))
      copies.append(pltpu.async_copy(eum_hbm.at[uidx_v.at[c]], um_v.at[sl], sem))
      copies.append(pltpu.async_copy(eig_hbm.at[iidx_v.at[c]], ig_v.at[sl], sem))
      copies.append(pltpu.async_copy(eim_hbm.at[iidx_v.at[c]], im_v.at[sl], sem))
    for cp in copies:
      cp.wait()
    out_sl = pl.ds(base, _BPW)
    pltpu.sync_copy(ug_v, ug_o.at[out_sl])
    pltpu.sync_copy(um_v, um_o.at[out_sl])
    pltpu.sync_copy(ig_v, ig_o.at[out_sl])
    pltpu.sync_copy(im_v, im_o.at[out_sl])

  return k(user_idx2d, item_idx2d, eug, eum, eig, eim)


def _tc_dense_body(ug_r, ig_r, um_r, im_r, gw_r, gb_r, w1a_r, w1b_r, b1_r,
                   w2_r, b2_r, w3_r, b3_r, w4_r, b4_r, out_r):
  ug = ug_r[...].astype(jnp.float32)
  ig = ig_r[...].astype(jnp.float32)
  gmf_logit = jnp.sum(ug * ig * gw_r[...], axis=1, keepdims=True) + gb_r[0, 0]
  h = jnp.maximum(
      jnp.dot(um_r[...].astype(jnp.float32), w1a_r[...],
              preferred_element_type=jnp.float32)
      + jnp.dot(im_r[...].astype(jnp.float32), w1b_r[...],
                preferred_element_type=jnp.float32)
      + b1_r[...], 0.0)
  h = jnp.maximum(
      jnp.dot(h, w2_r[...], preferred_element_type=jnp.float32) + b2_r[...],
      0.0)
  h = jnp.maximum(
      jnp.dot(h, w3_r[...], preferred_element_type=jnp.float32) + b3_r[...],
      0.0)
  mlp_logit = jnp.sum(h * w4_r[...], axis=1, keepdims=True) + b4_r[0, 0]
  out_r[...] = 0.5 * (jax.nn.sigmoid(gmf_logit) + jax.nn.sigmoid(mlp_logit))


def kernel(user_indices, item_indices, emb_user_gmf, emb_user_mlp,
           emb_item_gmf, emb_item_mlp, gmf_w, gmf_b, w1, b1, w2, b2, w3, b3,
           w4, b4):
  uidx = jnp.asarray(user_indices, jnp.int32).reshape(B // _CHUNK, _CHUNK)
  iidx = jnp.asarray(item_indices, jnp.int32).reshape(B // _CHUNK, _CHUNK)

  ug, um, ig, im = _sc_gather(
      uidx, iidx,
      emb_user_gmf.astype(jnp.bfloat16), emb_user_mlp.astype(jnp.bfloat16),
      emb_item_gmf.astype(jnp.bfloat16), emb_item_mlp.astype(jnp.bfloat16))

  gw = gmf_w.reshape(1, D)
  gb = gmf_b.reshape(1, 1)
  w1a = w1[:D]
  w1b = w1[D:]
  b1r = b1.reshape(1, -1)
  b2r = b2.reshape(1, -1)
  b3r = b3.reshape(1, -1)
  w4r = w4.reshape(1, -1)
  b4r = b4.reshape(1, 1)

  blk = 4096
  grid = B // blk

  def row_spec():
    return pl.BlockSpec((blk, D), lambda i: (i, 0))

  def full_spec(shape):
    return pl.BlockSpec(shape, lambda i: tuple(0 for _ in shape))

  out = pl.pallas_call(
      _tc_dense_body,
      grid=(grid,),
      in_specs=[
          row_spec(), row_spec(), row_spec(), row_spec(),
          full_spec(gw.shape), full_spec(gb.shape),
          full_spec(w1a.shape), full_spec(w1b.shape), full_spec(b1r.shape),
          full_spec(w2.shape), full_spec(b2r.shape),
          full_spec(w3.shape), full_spec(b3r.shape),
          full_spec(w4r.shape), full_spec(b4r.shape),
      ],
      out_specs=pl.BlockSpec((blk, 1), lambda i: (i, 0)),
      out_shape=jax.ShapeDtypeStruct((B, 1), jnp.float32),
  )(ug, ig, um, im, gw, gb, w1a, w1b, b1r, w2, b2r, w3, b3r, w4r, b4r)
  return out


# speedup vs baseline: 6.0672x; 1.3334x over previous
import functools

import jax
import jax.numpy as jnp
from jax import lax
from jax.experimental import pallas as pl
from jax.experimental.pallas import tpu as pltpu
from jax.experimental.pallas import tpu_sc as plsc

B = 16384
D = 32
W = 128

_NC, _NS = 2, 16
_NW = _NC * _NS
_BPW = B // _NW
_CHUNK = 128
_NCH = _BPW // _CHUNK


def _sc_gather(user_idx2d, item_idx2d, usr, itm):
  mesh = plsc.VectorSubcoreMesh(core_axis_name="c", subcore_axis_name="s")

  out_t = jax.ShapeDtypeStruct((B, W), jnp.float32)
  buf_t = pltpu.VMEM((_CHUNK, W), jnp.float32)

  @functools.partial(
      pl.kernel,
      mesh=mesh,
      out_type=[out_t, out_t],
      scratch_types=[
          pltpu.VMEM((_NCH, _CHUNK), jnp.int32),
          pltpu.VMEM((_NCH, _CHUNK), jnp.int32),
          buf_t, buf_t, buf_t, buf_t,
          pltpu.SemaphoreType.DMA,
      ],
  )
  def k(uidx_hbm, iidx_hbm, usr_hbm, itm_hbm, u_o, i_o,
        uidx_v, iidx_v, ub0, ub1, ib0, ib1, sem):
    wid = lax.axis_index("s") * _NC + lax.axis_index("c")
    base = wid * _BPW
    crow = wid * _NCH
    pltpu.sync_copy(uidx_hbm.at[pl.ds(crow, _NCH)], uidx_v)
    pltpu.sync_copy(iidx_hbm.at[pl.ds(crow, _NCH)], iidx_v)
    ubufs = (ub0, ub1)
    ibufs = (ib0, ib1)
    prev = None
    for c in range(_NCH):
      cu = pltpu.async_copy(usr_hbm.at[uidx_v.at[c]], ubufs[c % 2], sem)
      ci = pltpu.async_copy(itm_hbm.at[iidx_v.at[c]], ibufs[c % 2], sem)
      if prev is not None:
        pcu, pci, pc = prev
        pcu.wait()
        pci.wait()
        out_sl = pl.ds(base + pc * _CHUNK, _CHUNK)
        pltpu.sync_copy(ubufs[pc % 2], u_o.at[out_sl])
        pltpu.sync_copy(ibufs[pc % 2], i_o.at[out_sl])
      prev = (cu, ci, c)
    pcu, pci, pc = prev
    pcu.wait()
    pci.wait()
    out_sl = pl.ds(base + pc * _CHUNK, _CHUNK)
    pltpu.sync_copy(ubufs[pc % 2], u_o.at[out_sl])
    pltpu.sync_copy(ibufs[pc % 2], i_o.at[out_sl])

  return k(user_idx2d, item_idx2d, usr, itm)


def _tc_dense_body(u_r, i_r, gw_r, gb_r, w1a_r, w1b_r, b1_r,
                   w2_r, b2_r, w3_r, b3_r, w4_r, b4_r, out_r):
  u = u_r[...]
  it = i_r[...]
  ug = u[:, :D]
  um = u[:, D:2 * D]
  ig = it[:, :D]
  im = it[:, D:2 * D]
  gmf_logit = jnp.sum(ug * ig * gw_r[...], axis=1, keepdims=True) + gb_r[0, 0]
  h = jnp.maximum(
      jnp.dot(um, w1a_r[...], preferred_element_type=jnp.float32)
      + jnp.dot(im, w1b_r[...], preferred_element_type=jnp.float32)
      + b1_r[...], 0.0)
  h = jnp.maximum(
      jnp.dot(h, w2_r[...], preferred_element_type=jnp.float32) + b2_r[...],
      0.0)
  h = jnp.maximum(
      jnp.dot(h, w3_r[...], preferred_element_type=jnp.float32) + b3_r[...],
      0.0)
  mlp_logit = jnp.sum(h * w4_r[...], axis=1, keepdims=True) + b4_r[0, 0]
  out_r[...] = 0.5 * (jax.nn.sigmoid(gmf_logit) + jax.nn.sigmoid(mlp_logit))


def kernel(user_indices, item_indices, emb_user_gmf, emb_user_mlp,
           emb_item_gmf, emb_item_mlp, gmf_w, gmf_b, w1, b1, w2, b2, w3, b3,
           w4, b4):
  uidx = jnp.asarray(user_indices, jnp.int32).reshape(B // _CHUNK, _CHUNK)
  iidx = jnp.asarray(item_indices, jnp.int32).reshape(B // _CHUNK, _CHUNK)

  usr = jnp.pad(jnp.concatenate([emb_user_gmf, emb_user_mlp], axis=1),
                ((0, 0), (0, W - 2 * D)))
  itm = jnp.pad(jnp.concatenate([emb_item_gmf, emb_item_mlp], axis=1),
                ((0, 0), (0, W - 2 * D)))

  urows, irows = _sc_gather(uidx, iidx, usr, itm)

  gw = gmf_w.reshape(1, D)
  gb = gmf_b.reshape(1, 1)
  w1a = w1[:D]
  w1b = w1[D:]
  b1r = b1.reshape(1, -1)
  b2r = b2.reshape(1, -1)
  b3r = b3.reshape(1, -1)
  w4r = w4.reshape(1, -1)
  b4r = b4.reshape(1, 1)

  blk = 4096
  grid = B // blk

  def row_spec():
    return pl.BlockSpec((blk, W), lambda i: (i, 0))

  def full_spec(shape):
    return pl.BlockSpec(shape, lambda i: tuple(0 for _ in shape))

  out = pl.pallas_call(
      _tc_dense_body,
      grid=(grid,),
      in_specs=[
          row_spec(), row_spec(),
          full_spec(gw.shape), full_spec(gb.shape),
          full_spec(w1a.shape), full_spec(w1b.shape), full_spec(b1r.shape),
          full_spec(w2.shape), full_spec(b2r.shape),
          full_spec(w3.shape), full_spec(b3r.shape),
          full_spec(w4r.shape), full_spec(b4r.shape),
      ],
      out_specs=pl.BlockSpec((blk, 1), lambda i: (i, 0)),
      out_shape=jax.ShapeDtypeStruct((B, 1), jnp.float32),
  )(urows, irows, gw, gb, w1a, w1b, b1r, w2, b2r, w3, b3r, w4r, b4r)
  return out
